# Initial kernel scaffold; baseline (speedup 1.0000x reference)
import jax
import jax.numpy as jnp
from jax.experimental import pallas as pl


def kernel(x_product, x_warehouse, edge_index_pw, edge_index_wp, target_edge_index, target_edge_attr, params):
    raise NotImplementedError("write your pallas kernel here")


# speedup vs baseline: 9.1136x; 9.1136x over previous
import functools
import numpy as np
import jax
import jax.numpy as jnp
from jax.experimental import pallas as pl

N_NODES = 25000
E_EDGES = 400000
T_EDGES = 200000
HEADS = 4
DH = 32
MMBLK = 1000


def _mm_body(x_ref, w_ref, b_ref, o_ref, *, act):
    acc = jnp.dot(x_ref[...], w_ref[...], preferred_element_type=jnp.float32)
    acc = acc + b_ref[...]
    if act == "relu":
        acc = jnp.maximum(acc, 0.0)
    o_ref[...] = acc


def _mm(x, w, b, act="none", blk=MMBLK):
    m, kin = x.shape
    kout = w.shape[1]
    assert m % blk == 0
    grid = (m // blk,)
    return pl.pallas_call(
        functools.partial(_mm_body, act=act),
        grid=grid,
        in_specs=[
            pl.BlockSpec((blk, kin), lambda i: (i, 0)),
            pl.BlockSpec((kin, kout), lambda i: (0, 0)),
            pl.BlockSpec((1, kout), lambda i: (0, 0)),
        ],
        out_specs=pl.BlockSpec((blk, kout), lambda i: (i, 0)),
        out_shape=jax.ShapeDtypeStruct((m, kout), jnp.float32),
    )(x, w, b.reshape(1, kout))


def _gelu(x):
    return 0.5 * x * (1.0 + jax.lax.erf(x * np.float32(1.0 / np.sqrt(2.0))))


def _post_body(agg_ref, h_ref, wa_ref, ba_ref, g_ref, o_ref, *, act):
    g = _gelu(agg_ref[...])
    o = jnp.dot(g, wa_ref[...], preferred_element_type=jnp.float32)
    o = o + ba_ref[...] + g_ref[...] * h_ref[...]
    if act == "relu":
        o = jnp.maximum(o, 0.0)
    o_ref[...] = o


def _post(agg, h, wa, ba, gamma, act="none", blk=MMBLK):
    m, k = agg.shape
    grid = (m // blk,)
    return pl.pallas_call(
        functools.partial(_post_body, act=act),
        grid=grid,
        in_specs=[
            pl.BlockSpec((blk, k), lambda i: (i, 0)),
            pl.BlockSpec((blk, k), lambda i: (i, 0)),
            pl.BlockSpec((k, k), lambda i: (0, 0)),
            pl.BlockSpec((1, k), lambda i: (0, 0)),
            pl.BlockSpec((1, 1), lambda i: (0, 0)),
        ],
        out_specs=pl.BlockSpec((blk, k), lambda i: (i, 0)),
        out_shape=jax.ShapeDtypeStruct((m, k), jnp.float32),
    )(agg, h, wa, ba.reshape(1, k), gamma.reshape(1, 1))


def _dec_body(pg_ref, qg_ref, at_ref, w1c_ref, b1_ref, w2_ref, b2_ref, o_ref):
    s = pg_ref[...] + qg_ref[...] + b1_ref[...]
    s = s + jnp.dot(at_ref[...], w1c_ref[...], preferred_element_type=jnp.float32)
    s = jnp.maximum(s, 0.0)
    o_ref[...] = (jnp.dot(s, w2_ref[...], preferred_element_type=jnp.float32)
                  + b2_ref[...])


def _dec_final(pg, qg, attr, w1c, b1, w2, b2, blk=MMBLK):
    m, k = pg.shape
    ea = attr.shape[1]
    grid = (m // blk,)
    return pl.pallas_call(
        _dec_body,
        grid=grid,
        in_specs=[
            pl.BlockSpec((blk, k), lambda i: (i, 0)),
            pl.BlockSpec((blk, k), lambda i: (i, 0)),
            pl.BlockSpec((blk, ea), lambda i: (i, 0)),
            pl.BlockSpec((ea, k), lambda i: (0, 0)),
            pl.BlockSpec((1, k), lambda i: (0, 0)),
            pl.BlockSpec((k, 1), lambda i: (0, 0)),
            pl.BlockSpec((1, 1), lambda i: (0, 0)),
        ],
        out_specs=pl.BlockSpec((blk, 1), lambda i: (i, 0)),
        out_shape=jax.ShapeDtypeStruct((m, 1), jnp.float32),
    )(pg, qg, attr, w1c, b1.reshape(1, k), w2, b2.reshape(1, 1))


def _fold_rel(w, b, rel, scale=None):
    wf = jnp.einsum("ihd,hde->ihe", w.reshape(128, HEADS, DH), rel)
    bf = jnp.einsum("hd,hde->he", b.reshape(HEADS, DH), rel)
    if scale is not None:
        wf = wf * scale[None, :, None]
        bf = bf * scale[:, None]
    return wf.reshape(128, 128), bf.reshape(128)


def _layer_weights(params, c):
    out = {}
    rel_of_src = {"product": "pw", "warehouse": "wp"}
    for t in ("product", "warehouse"):
        r = rel_of_src[t]
        scale = params[c + "_prel_" + r] * np.float32(1.0 / np.sqrt(DH))
        wk, bk = _fold_rel(params[c + "_k_" + t + "_w"],
                           params[c + "_k_" + t + "_b"],
                           params[c + "_arel_" + r], scale)
        wv, bv = _fold_rel(params[c + "_v_" + t + "_w"],
                           params[c + "_v_" + t + "_b"],
                           params[c + "_mrel_" + r])
        wcat = jnp.concatenate(
            [params[c + "_q_" + t + "_w"], wk, wv], axis=1)
        bcat = jnp.concatenate(
            [params[c + "_q_" + t + "_b"], bk, bv], axis=0)
        out[t] = (wcat, bcat)
    return out


def _edge_pass(k_rel_s, q_d, v_rel_s, src, dst):
    kg = k_rel_s[src]
    qg = q_d[dst]
    alpha = (kg * qg).reshape(-1, HEADS, DH).sum(-1)
    amax = jax.ops.segment_max(alpha, dst, num_segments=N_NODES)
    amax = jnp.where(jnp.isfinite(amax), amax, 0.0)
    ex = jnp.exp(alpha - amax[dst])
    den = jax.ops.segment_sum(ex, dst, num_segments=N_NODES)
    msg = v_rel_s[src].reshape(-1, HEADS, DH) * ex[:, :, None]
    num = jax.ops.segment_sum(msg.reshape(-1, 128), dst, num_segments=N_NODES)
    return num / (den + 1e-16).repeat(DH, axis=1)


def _hgt_layer(c, h, e_pw, e_wp, params):
    lw = _layer_weights(params, c)
    proj = {}
    for t in ("product", "warehouse"):
        w, b = lw[t]
        z = _mm(h[t], w, b)
        proj[t] = (z[:, :128], z[:, 128:256], z[:, 256:384])
    agg = {}
    for r, s, d, ei in (("pw", "product", "warehouse", e_pw),
                        ("wp", "warehouse", "product", e_wp)):
        agg[d] = _edge_pass(proj[s][1], proj[d][0], proj[s][2], ei[0], ei[1])
    out = {}
    for t in ("product", "warehouse"):
        beta = jax.nn.sigmoid(params[c + "_skip_" + t])
        wa = params[c + "_a_" + t + "_w"] * beta
        ba = params[c + "_a_" + t + "_b"] * beta
        out[t] = _post(agg[t], h[t], wa, ba, 1.0 - beta,
                       act="relu" if c == "c1" else "none")
    return out


def kernel(x_product, x_warehouse, edge_index_pw, edge_index_wp,
           target_edge_index, target_edge_attr, params):
    h = {
        "product": _mm(x_product, params["in_product_w"],
                       params["in_product_b"], act="relu"),
        "warehouse": _mm(x_warehouse, params["in_warehouse_w"],
                         params["in_warehouse_b"], act="relu"),
    }
    h = _hgt_layer("c1", h, edge_index_pw, edge_index_wp, params)
    h = _hgt_layer("c2", h, edge_index_pw, edge_index_wp, params)

    w1 = params["dec1_w"]
    zeros = jnp.zeros((128,), jnp.float32)
    p_arr = _mm(h["product"], w1[:128], zeros)
    q_arr = _mm(h["warehouse"], w1[128:256], zeros)
    src, dst = target_edge_index[0], target_edge_index[1]
    out = _dec_final(p_arr[src], q_arr[dst], target_edge_attr,
                     w1[256:260], params["dec1_b"],
                     params["dec2_w"], params["dec2_b"])
    return out.reshape(-1)


# speedup vs baseline: 12.6545x; 1.3885x over previous
import functools
import numpy as np
import jax
import jax.numpy as jnp
from jax import lax
from jax.experimental import pallas as pl
from jax.experimental.pallas import tpu as pltpu, tpu_sc as plsc

N_NODES = 25000
E_EDGES = 400000
T_EDGES = 200000
HEADS = 4
DH = 32
MMBLK = 1000

SC_NC = 2
SC_NS = 16
SC_NW = SC_NC * SC_NS
GCH = 640
N_CHUNKS = E_EDGES // GCH
CH_PER_W = -(-N_CHUNKS // SC_NW)


def _gather3_body(tk, ik, tq, iq, tv, iv, kg, qg, vg, idx_v, rows_v, sem):
    w = lax.axis_index("s") * SC_NC + lax.axis_index("c")

    def chunk(j, carry):
        cw = w + j * SC_NW

        @pl.when(cw < N_CHUNKS)
        def _():
            off = pl.multiple_of(cw * GCH, GCH)
            for tab, ind, out in ((tk, ik, kg), (tq, iq, qg), (tv, iv, vg)):
                pltpu.sync_copy(ind.at[pl.ds(off, GCH)], idx_v)
                pltpu.async_copy(tab.at[idx_v], rows_v, sem).wait()
                pltpu.sync_copy(rows_v, out.at[pl.ds(off, GCH)])
        return carry

    lax.fori_loop(0, CH_PER_W, chunk, 0)


def _gather3(tk, ik, tq, iq, tv, iv):
    fn = pl.kernel(
        _gather3_body,
        out_type=[jax.ShapeDtypeStruct((E_EDGES, 128), jnp.float32)] * 3,
        mesh=plsc.VectorSubcoreMesh(core_axis_name="c", subcore_axis_name="s"),
        scratch_types=[
            pltpu.VMEM((GCH,), jnp.int32),
            pltpu.VMEM((GCH, 128), jnp.float32),
            pltpu.SemaphoreType.DMA,
        ],
    )
    return fn(tk, ik, tq, iq, tv, iv)


def _alpha_body(kg_ref, qg_ref, o_ref):
    p = kg_ref[...] * qg_ref[...]
    o_ref[...] = p.reshape(-1, HEADS, DH).sum(-1)


def _alpha_tc(kg, qg, blk=2000):
    m = kg.shape[0]
    return pl.pallas_call(
        _alpha_body,
        grid=(m // blk,),
        in_specs=[pl.BlockSpec((blk, 128), lambda i: (i, 0)),
                  pl.BlockSpec((blk, 128), lambda i: (i, 0))],
        out_specs=pl.BlockSpec((blk, HEADS), lambda i: (i, 0)),
        out_shape=jax.ShapeDtypeStruct((m, HEADS), jnp.float32),
    )(kg, qg)


def _msg_body(vg_ref, ex_ref, o_ref):
    v = vg_ref[...].reshape(-1, HEADS, DH)
    o_ref[...] = (v * ex_ref[...][:, :, None]).reshape(-1, 128)


def _msg_tc(vg, ex, blk=2000):
    m = vg.shape[0]
    return pl.pallas_call(
        _msg_body,
        grid=(m // blk,),
        in_specs=[pl.BlockSpec((blk, 128), lambda i: (i, 0)),
                  pl.BlockSpec((blk, HEADS), lambda i: (i, 0))],
        out_specs=pl.BlockSpec((blk, 128), lambda i: (i, 0)),
        out_shape=jax.ShapeDtypeStruct((m, 128), jnp.float32),
    )(vg, ex)


def _mm_body(x_ref, w_ref, b_ref, o_ref, *, act):
    acc = jnp.dot(x_ref[...], w_ref[...], preferred_element_type=jnp.float32)
    acc = acc + b_ref[...]
    if act == "relu":
        acc = jnp.maximum(acc, 0.0)
    o_ref[...] = acc


def _mm(x, w, b, act="none", blk=MMBLK):
    m, kin = x.shape
    kout = w.shape[1]
    assert m % blk == 0
    grid = (m // blk,)
    return pl.pallas_call(
        functools.partial(_mm_body, act=act),
        grid=grid,
        in_specs=[
            pl.BlockSpec((blk, kin), lambda i: (i, 0)),
            pl.BlockSpec((kin, kout), lambda i: (0, 0)),
            pl.BlockSpec((1, kout), lambda i: (0, 0)),
        ],
        out_specs=pl.BlockSpec((blk, kout), lambda i: (i, 0)),
        out_shape=jax.ShapeDtypeStruct((m, kout), jnp.float32),
    )(x, w, b.reshape(1, kout))


def _gelu(x):
    return 0.5 * x * (1.0 + jax.lax.erf(x * np.float32(1.0 / np.sqrt(2.0))))


def _post_body(agg_ref, h_ref, wa_ref, ba_ref, g_ref, o_ref, *, act):
    g = _gelu(agg_ref[...])
    o = jnp.dot(g, wa_ref[...], preferred_element_type=jnp.float32)
    o = o + ba_ref[...] + g_ref[...] * h_ref[...]
    if act == "relu":
        o = jnp.maximum(o, 0.0)
    o_ref[...] = o


def _post(agg, h, wa, ba, gamma, act="none", blk=MMBLK):
    m, k = agg.shape
    grid = (m // blk,)
    return pl.pallas_call(
        functools.partial(_post_body, act=act),
        grid=grid,
        in_specs=[
            pl.BlockSpec((blk, k), lambda i: (i, 0)),
            pl.BlockSpec((blk, k), lambda i: (i, 0)),
            pl.BlockSpec((k, k), lambda i: (0, 0)),
            pl.BlockSpec((1, k), lambda i: (0, 0)),
            pl.BlockSpec((1, 1), lambda i: (0, 0)),
        ],
        out_specs=pl.BlockSpec((blk, k), lambda i: (i, 0)),
        out_shape=jax.ShapeDtypeStruct((m, k), jnp.float32),
    )(agg, h, wa, ba.reshape(1, k), gamma.reshape(1, 1))


def _dec_body(pg_ref, qg_ref, at_ref, w1c_ref, b1_ref, w2_ref, b2_ref, o_ref):
    s = pg_ref[...] + qg_ref[...] + b1_ref[...]
    s = s + jnp.dot(at_ref[...], w1c_ref[...], preferred_element_type=jnp.float32)
    s = jnp.maximum(s, 0.0)
    o_ref[...] = (jnp.dot(s, w2_ref[...], preferred_element_type=jnp.float32)
                  + b2_ref[...])


def _dec_final(pg, qg, attr, w1c, b1, w2, b2, blk=MMBLK):
    m, k = pg.shape
    ea = attr.shape[1]
    grid = (m // blk,)
    return pl.pallas_call(
        _dec_body,
        grid=grid,
        in_specs=[
            pl.BlockSpec((blk, k), lambda i: (i, 0)),
            pl.BlockSpec((blk, k), lambda i: (i, 0)),
            pl.BlockSpec((blk, ea), lambda i: (i, 0)),
            pl.BlockSpec((ea, k), lambda i: (0, 0)),
            pl.BlockSpec((1, k), lambda i: (0, 0)),
            pl.BlockSpec((k, 1), lambda i: (0, 0)),
            pl.BlockSpec((1, 1), lambda i: (0, 0)),
        ],
        out_specs=pl.BlockSpec((blk, 1), lambda i: (i, 0)),
        out_shape=jax.ShapeDtypeStruct((m, 1), jnp.float32),
    )(pg, qg, attr, w1c, b1.reshape(1, k), w2, b2.reshape(1, 1))


def _fold_rel(w, b, rel, scale=None):
    wf = jnp.einsum("ihd,hde->ihe", w.reshape(128, HEADS, DH), rel)
    bf = jnp.einsum("hd,hde->he", b.reshape(HEADS, DH), rel)
    if scale is not None:
        wf = wf * scale[None, :, None]
        bf = bf * scale[:, None]
    return wf.reshape(128, 128), bf.reshape(128)


def _layer_weights(params, c):
    out = {}
    rel_of_src = {"product": "pw", "warehouse": "wp"}
    for t in ("product", "warehouse"):
        r = rel_of_src[t]
        scale = params[c + "_prel_" + r] * np.float32(1.0 / np.sqrt(DH))
        wk, bk = _fold_rel(params[c + "_k_" + t + "_w"],
                           params[c + "_k_" + t + "_b"],
                           params[c + "_arel_" + r], scale)
        wv, bv = _fold_rel(params[c + "_v_" + t + "_w"],
                           params[c + "_v_" + t + "_b"],
                           params[c + "_mrel_" + r])
        wcat = jnp.concatenate(
            [params[c + "_q_" + t + "_w"], wk, wv], axis=1)
        bcat = jnp.concatenate(
            [params[c + "_q_" + t + "_b"], bk, bv], axis=0)
        out[t] = (wcat, bcat)
    return out


def _edge_pass(k_rel_s, q_d, v_rel_s, src, dst):
    kg, qg, vg = _gather3(k_rel_s, src, q_d, dst, v_rel_s, src)
    alpha = _alpha_tc(kg, qg)
    amax = jax.ops.segment_max(alpha, dst, num_segments=N_NODES)
    amax = jnp.where(jnp.isfinite(amax), amax, 0.0)
    ex = jnp.exp(alpha - amax[dst])
    den = jax.ops.segment_sum(ex, dst, num_segments=N_NODES)
    num = jax.ops.segment_sum(_msg_tc(vg, ex), dst, num_segments=N_NODES)
    return num / (den + 1e-16).repeat(DH, axis=1)


def _hgt_layer(c, h, e_pw, e_wp, params):
    lw = _layer_weights(params, c)
    proj = {}
    for t in ("product", "warehouse"):
        w, b = lw[t]
        z = _mm(h[t], w, b)
        proj[t] = (z[:, :128], z[:, 128:256], z[:, 256:384])
    agg = {}
    for r, s, d, ei in (("pw", "product", "warehouse", e_pw),
                        ("wp", "warehouse", "product", e_wp)):
        agg[d] = _edge_pass(proj[s][1], proj[d][0], proj[s][2], ei[0], ei[1])
    out = {}
    for t in ("product", "warehouse"):
        beta = jax.nn.sigmoid(params[c + "_skip_" + t])
        wa = params[c + "_a_" + t + "_w"] * beta
        ba = params[c + "_a_" + t + "_b"] * beta
        out[t] = _post(agg[t], h[t], wa, ba, 1.0 - beta,
                       act="relu" if c == "c1" else "none")
    return out


def kernel(x_product, x_warehouse, edge_index_pw, edge_index_wp,
           target_edge_index, target_edge_attr, params):
    h = {
        "product": _mm(x_product, params["in_product_w"],
                       params["in_product_b"], act="relu"),
        "warehouse": _mm(x_warehouse, params["in_warehouse_w"],
                         params["in_warehouse_b"], act="relu"),
    }
    h = _hgt_layer("c1", h, edge_index_pw, edge_index_wp, params)
    h = _hgt_layer("c2", h, edge_index_pw, edge_index_wp, params)

    w1 = params["dec1_w"]
    zeros = jnp.zeros((128,), jnp.float32)
    p_arr = _mm(h["product"], w1[:128], zeros)
    q_arr = _mm(h["warehouse"], w1[128:256], zeros)
    src, dst = target_edge_index[0], target_edge_index[1]
    out = _dec_final(p_arr[src], q_arr[dst], target_edge_attr,
                     w1[256:260], params["dec1_b"],
                     params["dec2_w"], params["dec2_b"])
    return out.reshape(-1)
